# baseline (speedup 1.0000x reference)
import jax
import jax.numpy as jnp
from jax.experimental import pallas as pl
from jax.experimental.pallas import tpu as pltpu

M = 2048
N = 32768
SPARSITY = 0.25
SLOPE1 = 10.0
SLOPE2 = 10.0
K_LO = 24575
BM = 64
N_BLK = M // BM


def _sigmoid(x):
    return 1.0 / (1.0 + jnp.exp(-x))


def _body(l_ref, s_ref, prob_ref, out_ref, buf_ref, sem):
    if True:
        probs = _sigmoid(SLOPE1 * l_ref[...])
        x_bar = jnp.mean(probs)
        r = SPARSITY / x_bar
        beta = (1.0 - SPARSITY) / (1.0 - x_bar)
        le = (r <= 1.0).astype(probs.dtype)
        pm = le * probs * r + (1.0 - le) * (1.0 - (1.0 - probs) * beta)
        prob_ref[...] = pm

        im = _sigmoid(SLOPE2 * (pm - s_ref[...]))
        xi = jax.lax.bitcast_convert_type(im, jnp.int32)

        def bit_step(i, acc):
            cand = acc + (jnp.int32(1) << (30 - i))
            cnt = jnp.sum((xi < cand).astype(jnp.int32))
            return jnp.where(cnt <= K_LO, cand, acc)

        acc = jax.lax.fori_loop(0, 31, bit_step, jnp.int32(0))
        s_lo = jax.lax.bitcast_convert_type(acc, jnp.float32)
        cnt_le = jnp.sum((xi <= acc).astype(jnp.int32))
        s_hi_next = jnp.min(jnp.where(xi > acc, im, jnp.float32(2.0)))
        s_hi = jnp.where(cnt_le >= K_LO + 2, s_lo, s_hi_next)
        thresh = s_lo * jnp.float32(1.0 - SPARSITY) + s_hi * jnp.float32(SPARSITY)
        mask = (im >= thresh).astype(jnp.float32)
        buf_ref[...] = jnp.broadcast_to(mask, (BM, N))

    for i in range(N_BLK):
        pltpu.make_async_copy(buf_ref, out_ref.at[pl.ds(i * BM, BM)], sem).start(
            priority=i % 2
        )
    for i in range(N_BLK):
        pltpu.make_async_copy(buf_ref, out_ref.at[pl.ds(i * BM, BM)], sem).wait()


def kernel(logits, sample_mask):
    l1 = logits.reshape(1, N)
    s1 = sample_mask.reshape(1, N)
    prob1, out = pl.pallas_call(
        _body,
        out_shape=[
            jax.ShapeDtypeStruct((1, N), jnp.float32),
            jax.ShapeDtypeStruct((M, N), jnp.float32),
        ],
        in_specs=[
            pl.BlockSpec(memory_space=pltpu.MemorySpace.VMEM),
            pl.BlockSpec(memory_space=pltpu.MemorySpace.VMEM),
        ],
        out_specs=[
            pl.BlockSpec(memory_space=pltpu.MemorySpace.VMEM),
            pl.BlockSpec(memory_space=pl.ANY),
        ],
        scratch_shapes=[
            pltpu.VMEM((BM, N), jnp.float32),
            pltpu.SemaphoreType.DMA,
        ],
    )(l1, s1)
    return (prob1.reshape(1, 1, N), out.reshape(1, M, N))


# speedup vs baseline: 1.0120x; 1.0120x over previous
import jax
import jax.numpy as jnp
from jax.experimental import pallas as pl
from jax.experimental.pallas import tpu as pltpu

M = 2048
N = 32768
SPARSITY = 0.25
SLOPE1 = 10.0
SLOPE2 = 10.0
K_LO = 24575
BM = 64
N_BLK = M // BM


def _sigmoid(x):
    return 1.0 / (1.0 + jnp.exp(-x))


def _body(l_ref, s_ref, prob_ref, out_ref, mask_ref):
    step = pl.program_id(0)

    @pl.when(step == 0)
    def _phase_a():
        probs = _sigmoid(SLOPE1 * l_ref[...])
        x_bar = jnp.mean(probs)
        r = SPARSITY / x_bar
        beta = (1.0 - SPARSITY) / (1.0 - x_bar)
        le = (r <= 1.0).astype(probs.dtype)
        pm = le * probs * r + (1.0 - le) * (1.0 - (1.0 - probs) * beta)
        prob_ref[...] = pm

        im = _sigmoid(SLOPE2 * (pm - s_ref[...]))
        xi = jax.lax.bitcast_convert_type(im, jnp.int32)

        def bit_step(i, acc):
            cand = acc + (jnp.int32(1) << (30 - i))
            cnt = jnp.sum((xi < cand).astype(jnp.int32))
            return jnp.where(cnt <= K_LO, cand, acc)

        acc = jax.lax.fori_loop(0, 31, bit_step, jnp.int32(0))
        s_lo = jax.lax.bitcast_convert_type(acc, jnp.float32)
        cnt_le = jnp.sum((xi <= acc).astype(jnp.int32))
        s_hi_next = jnp.min(jnp.where(xi > acc, im, jnp.float32(2.0)))
        s_hi = jnp.where(cnt_le >= K_LO + 2, s_lo, s_hi_next)
        thresh = s_lo * jnp.float32(1.0 - SPARSITY) + s_hi * jnp.float32(SPARSITY)
        mask_ref[...] = (im >= thresh).astype(jnp.float32)

    out_ref[...] = jnp.broadcast_to(mask_ref[...], (BM, N))


def kernel(logits, sample_mask):
    l1 = logits.reshape(1, N)
    s1 = sample_mask.reshape(1, N)
    prob1, out = pl.pallas_call(
        _body,
        grid=(N_BLK,),
        out_shape=[
            jax.ShapeDtypeStruct((1, N), jnp.float32),
            jax.ShapeDtypeStruct((M, N), jnp.float32),
        ],
        in_specs=[
            pl.BlockSpec((1, N), lambda i: (0, 0)),
            pl.BlockSpec((1, N), lambda i: (0, 0)),
        ],
        out_specs=[
            pl.BlockSpec((1, N), lambda i: (0, 0)),
            pl.BlockSpec((BM, N), lambda i: (i, 0)),
        ],
        scratch_shapes=[
            pltpu.VMEM((1, N), jnp.float32),
        ],
    )(l1, s1)
    return (prob1.reshape(1, 1, N), out.reshape(1, M, N))
